# baseline (speedup 1.0000x reference)
import functools

import jax
import jax.numpy as jnp
from jax import lax
from jax.experimental import pallas as pl
from jax.experimental.pallas import tpu as pltpu
from jax.experimental.pallas import tpu_sc as plsc

NC = 2
NS = 16
NW = NC * NS
CHUNK = 128
NBUF = 8


def _embed_kernel(n_fields, vocab, dim, batch):
    n_rows = n_fields * batch
    chunks_per_field = batch // CHUNK
    per_w = (n_rows // CHUNK) // NW
    n_groups = per_w // NBUF
    mesh = plsc.VectorSubcoreMesh(core_axis_name="c", subcore_axis_name="s")

    @functools.partial(
        pl.kernel,
        mesh=mesh,
        compiler_params=pltpu.CompilerParams(use_tc_tiling_on_sc=False),
        out_type=jax.ShapeDtypeStruct((batch, n_fields * dim), jnp.float32),
        scratch_types=(
            [pltpu.VMEM((NBUF, CHUNK), jnp.int32)]
            + [pltpu.VMEM((CHUNK, dim), jnp.float32) for _ in range(NBUF)]
            + [pltpu.SemaphoreType.DMA for _ in range(3 * NBUF)]
        ),
    )
    def k(tables_hbm, featsT_hbm, out_hbm, idx_v, *bufs_sems):
        rows = bufs_sems[:NBUF]
        gsem = bufs_sems[NBUF:2 * NBUF]
        osem = bufs_sems[2 * NBUF:3 * NBUF]
        isem = bufs_sems[3 * NBUF:]

        wid = lax.axis_index("s") * NC + lax.axis_index("c")
        c0 = wid * per_w

        def stage_idx(k_, b):
            c = c0 + k_
            f = c // chunks_per_field
            b0 = (c % chunks_per_field) * CHUNK
            pltpu.make_async_copy(
                featsT_hbm.at[f, pl.ds(b0, CHUNK)], idx_v.at[b], isem[b],
            ).start()

        def idx_wait(b):
            pltpu.make_async_copy(
                featsT_hbm.at[0, pl.ds(0, CHUNK)], idx_v.at[b], isem[b],
            ).wait()

        def out_slice(k_):
            c = c0 + k_
            f = c // chunks_per_field
            b0 = (c % chunks_per_field) * CHUNK
            return out_hbm.at[pl.ds(b0, CHUNK), pl.ds(f * dim, dim)]

        def gather(k_, b):
            c = c0 + k_
            f = c // chunks_per_field
            pltpu.make_async_copy(
                tables_hbm.at[f].at[idx_v.at[b]], rows[b], gsem[b],
            ).start()

        def drain_and_put(k_, b):
            pltpu.make_async_copy(
                tables_hbm.at[0].at[idx_v.at[b]], rows[b], gsem[b],
            ).wait()
            pltpu.make_async_copy(rows[b], out_slice(k_), osem[b]).start()
            @pl.when(k_ + NBUF < per_w)
            def _():
                stage_idx(k_ + NBUF, b)

        def out_wait(k_, b):
            pltpu.make_async_copy(rows[b], out_slice(k_), osem[b]).wait()

        for b in range(NBUF):
            stage_idx(b, b)

        def group(g, carry):
            for b in range(NBUF):
                k_ = g * NBUF + b

                @pl.when(g > 0)
                def _():
                    out_wait(k_ - NBUF, b)

                idx_wait(b)
                gather(k_, b)
            for b in range(NBUF):
                drain_and_put(g * NBUF + b, b)
            return carry

        lax.fori_loop(0, n_groups, group, 0)
        for b in range(NBUF):
            out_wait((n_groups - 1) * NBUF + b, b)

    return k


def kernel(features, tables):
    b, f = features.shape
    f2, vocab, dim = tables.shape
    assert f == f2
    n_chunks = b * f // CHUNK
    assert b % CHUNK == 0 and n_chunks % (NW * NBUF) == 0

    feats_t = features.astype(jnp.int32).T
    return _embed_kernel(f, vocab, dim, b)(tables, feats_t)
